# baseline (speedup 1.0000x reference)
import dataclasses
import functools

import jax
import jax.numpy as jnp
from jax import lax
from jax.experimental import pallas as pl
from jax.experimental.pallas import tpu as pltpu
from jax.experimental.pallas import tpu_sc as plsc

NC = 2
NS = 16
NW = NC * NS
CHUNK = 128
DEG_W = 16


def _ceil_to(a, m):
    return (a + m - 1) // m * m


def _tc_matmul(x, W):
    n, d_in = x.shape
    d_out = W.shape[0]
    bn = 1024

    def body(x_ref, w_ref, o_ref):
        o_ref[...] = lax.dot_general(
            x_ref[...], w_ref[...], (((1,), (1,)), ((), ())),
            preferred_element_type=jnp.float32,
            precision=lax.Precision.HIGHEST)

    return pl.pallas_call(
        body,
        grid=(n // bn,),
        in_specs=[pl.BlockSpec((bn, d_in), lambda i: (i, 0)),
                  pl.BlockSpec((d_out, d_in), lambda i: (0, 0))],
        out_specs=pl.BlockSpec((bn, d_out), lambda i: (i, 0)),
        out_shape=jax.ShapeDtypeStruct((n, d_out), jnp.float32),
    )(x, W)


def _deg_col(d_ref):
    ones = jnp.ones((NW, 1), jnp.float32)
    return lax.dot_general(d_ref[...], ones, (((0,), (0,)), ((), ())),
                           preferred_element_type=jnp.float32,
                           precision=lax.Precision.HIGHEST)


def _tc_prescale(h, degp):
    n, d = h.shape
    bn = 1024

    def body(h_ref, d_ref, o_ref):
        deg = _deg_col(d_ref) + 1.0
        o_ref[...] = h_ref[...] * lax.rsqrt(deg)

    return pl.pallas_call(
        body,
        grid=(n // bn,),
        in_specs=[pl.BlockSpec((bn, d), lambda i: (i, 0)),
                  pl.BlockSpec((NW, bn), lambda i: (0, i))],
        out_specs=pl.BlockSpec((bn, d), lambda i: (i, 0)),
        out_shape=jax.ShapeDtypeStruct((n, d), jnp.float32),
    )(h, degp)


def _tc_final(acc0, acc1, h, degp, b):
    n, d = h.shape
    bn = 1024

    def body(a0_ref, a1_ref, h_ref, d_ref, b_ref, o_ref):
        dis = lax.rsqrt(_deg_col(d_ref) + 1.0)
        o_ref[...] = (dis * (a0_ref[...] + a1_ref[...])
                      + (dis * dis) * h_ref[...] + b_ref[...])

    return pl.pallas_call(
        body,
        grid=(n // bn,),
        in_specs=[pl.BlockSpec((bn, d), lambda i: (i, 0)),
                  pl.BlockSpec((bn, d), lambda i: (i, 0)),
                  pl.BlockSpec((bn, d), lambda i: (i, 0)),
                  pl.BlockSpec((NW, bn), lambda i: (0, i)),
                  pl.BlockSpec((1, d), lambda i: (0, 0))],
        out_specs=pl.BlockSpec((bn, d), lambda i: (i, 0)),
        out_shape=jax.ShapeDtypeStruct((n, d), jnp.float32),
    )(acc0, acc1, h, degp, b.reshape(1, d))


def _sc_degree(dst_p, zeros_tab, n_nodes):
    ep = dst_p.shape[0]
    per_w = ep // NW
    per_sub = _ceil_to((n_nodes + 1 + NS - 1) // NS, CHUNK)
    n_pad = per_sub * NS
    mesh = plsc.VectorSubcoreMesh(core_axis_name="c", subcore_axis_name="s")

    cp = pltpu.CompilerParams()
    if "needs_layout_passes" in pltpu.CompilerParams.__dataclass_fields__:
        cp = dataclasses.replace(cp, needs_layout_passes=False)

    @functools.partial(
        pl.kernel, mesh=mesh,
        out_type=jax.ShapeDtypeStruct((NW, n_pad), jnp.float32),
        compiler_params=cp,
        scratch_types=[
            pltpu.VMEM((per_w,), jnp.int32),
            pltpu.VMEM((n_pad,), jnp.float32),
        ])
    def deg_kernel(dst_hbm, zeros_hbm, out_hbm, idx_v, tab_v):
        cid = lax.axis_index("c")
        sid = lax.axis_index("s")
        wid = cid * NS + sid
        pltpu.sync_copy(dst_hbm.at[pl.ds(wid * per_w, per_w)], idx_v)
        pltpu.sync_copy(zeros_hbm, tab_v)
        ones16 = jnp.ones((16,), jnp.float32)

        @pl.loop(0, per_w // 16)
        def _(k):
            idx = idx_v[pl.ds(k * 16, 16)]
            plsc.addupdate_scatter(tab_v, [idx], ones16)

        pltpu.sync_copy(tab_v, out_hbm.at[wid])

    return deg_kernel(dst_p, zeros_tab)


def _sc_aggregate(hs, src_p, dst_p, zeros_agg, n_nodes):
    ep = src_p.shape[0]
    d = hs.shape[1]
    cpw = ep // (NW * CHUNK)
    per_w = cpw * CHUNK
    per_sub = _ceil_to((n_nodes + 1 + NS - 1) // NS, CHUNK)
    acc_rows = per_sub * NS
    mesh = plsc.VectorSubcoreMesh(core_axis_name="c", subcore_axis_name="s")

    half = cpw // 2

    @functools.partial(
        pl.kernel, mesh=mesh,
        out_type=jax.ShapeDtypeStruct((NC, acc_rows, d), jnp.float32),
        scratch_types=[
            pltpu.VMEM((half, CHUNK), jnp.int32),
            pltpu.VMEM((half, CHUNK), jnp.int32),
            pltpu.VMEM((CHUNK, d), jnp.float32),
            pltpu.VMEM_SHARED((acc_rows, d), jnp.float32),
        ])
    def agg_kernel(hs_hbm, src_hbm, dst_hbm, zeros_hbm, out_hbm,
                   src_v, dst_v, rows, acc_sh):
        cid = lax.axis_index("c")
        sid = lax.axis_index("s")
        wid = cid * NS + sid

        pltpu.sync_copy(zeros_hbm, acc_sh.at[pl.ds(sid * per_sub, per_sub)])

        plsc.subcore_barrier()

        def scatter(c, rows):
            pltpu.sync_copy(rows, acc_sh.at[dst_v.at[c]], add=True)

        for hlf in range(2):
            pltpu.sync_copy(src_hbm.at[pl.ds(wid * cpw + hlf * half, half)],
                            src_v)
            pltpu.sync_copy(dst_hbm.at[pl.ds(wid * cpw + hlf * half, half)],
                            dst_v)

            @pl.loop(0, half)
            def _(i):
                pltpu.sync_copy(hs_hbm.at[src_v.at[i]], rows)
                scatter(i, rows)

        plsc.subcore_barrier()
        pltpu.sync_copy(acc_sh.at[pl.ds(sid * per_sub, per_sub)],
                        out_hbm.at[cid, pl.ds(sid * per_sub, per_sub)])

    return agg_kernel(hs, src_p.reshape(-1, CHUNK), dst_p.reshape(-1, CHUNK),
                      zeros_agg)


def kernel(x, edge_index, W, b):
    n, _ = x.shape
    e = edge_index.shape[1]
    src = edge_index[0].astype(jnp.int32)
    dst = edge_index[1].astype(jnp.int32)

    ep = _ceil_to(e, NW * CHUNK * 8)
    per_sub = _ceil_to((n + 1 + NS - 1) // NS, CHUNK)
    acc_rows = per_sub * NS
    pad_dst = n + jnp.arange(ep - e, dtype=jnp.int32) % (acc_rows - n)
    src_p = jnp.concatenate([src, jnp.zeros((ep - e,), jnp.int32)])
    dst_p = jnp.concatenate([dst, pad_dst])

    zeros_tab = jnp.zeros((acc_rows,), jnp.float32)
    zeros_agg = jnp.zeros((per_sub, W.shape[0]), jnp.float32)

    x_pad = jnp.concatenate(
        [x, jnp.zeros((acc_rows - n, x.shape[1]), x.dtype)])

    degp = _sc_degree(dst_p, zeros_tab, n)
    h = _tc_matmul(x_pad, W)
    hs = _tc_prescale(h, degp)
    acc_part = _sc_aggregate(hs, src_p, dst_p, zeros_agg, n)
    out = _tc_final(acc_part[0], acc_part[1], h, degp, b)
    return out[:n]


# speedup vs baseline: 1.0094x; 1.0094x over previous
import dataclasses
import functools

import jax
import jax.numpy as jnp
from jax import lax
from jax.experimental import pallas as pl
from jax.experimental.pallas import tpu as pltpu
from jax.experimental.pallas import tpu_sc as plsc

NC = 2
NS = 16
NW = NC * NS
CHUNK = 128
DEG_W = 16


def _ceil_to(a, m):
    return (a + m - 1) // m * m


def _tc_matmul(x, W):
    n, d_in = x.shape
    d_out = W.shape[0]
    bn = 1024

    def body(x_ref, w_ref, o_ref):
        o_ref[...] = lax.dot_general(
            x_ref[...], w_ref[...], (((1,), (1,)), ((), ())),
            preferred_element_type=jnp.float32,
            precision=lax.Precision.HIGHEST)

    return pl.pallas_call(
        body,
        grid=(n // bn,),
        in_specs=[pl.BlockSpec((bn, d_in), lambda i: (i, 0)),
                  pl.BlockSpec((d_out, d_in), lambda i: (0, 0))],
        out_specs=pl.BlockSpec((bn, d_out), lambda i: (i, 0)),
        out_shape=jax.ShapeDtypeStruct((n, d_out), jnp.float32),
    )(x, W)


def _deg_col(d_ref):
    ones = jnp.ones((NW, 1), jnp.float32)
    return lax.dot_general(d_ref[...], ones, (((0,), (0,)), ((), ())),
                           preferred_element_type=jnp.float32,
                           precision=lax.Precision.HIGHEST)


def _tc_prescale(h, degp):
    n, d = h.shape
    bn = 1024

    def body(h_ref, d_ref, o_ref):
        deg = _deg_col(d_ref) + 1.0
        o_ref[...] = h_ref[...] * lax.rsqrt(deg)

    return pl.pallas_call(
        body,
        grid=(n // bn,),
        in_specs=[pl.BlockSpec((bn, d), lambda i: (i, 0)),
                  pl.BlockSpec((NW, bn), lambda i: (0, i))],
        out_specs=pl.BlockSpec((bn, d), lambda i: (i, 0)),
        out_shape=jax.ShapeDtypeStruct((n, d), jnp.float32),
    )(h, degp)


def _tc_final(acc0, acc1, h, degp, b):
    n, d = h.shape
    bn = 1024

    def body(a0_ref, a1_ref, h_ref, d_ref, b_ref, o_ref):
        dis = lax.rsqrt(_deg_col(d_ref) + 1.0)
        o_ref[...] = (dis * (a0_ref[...] + a1_ref[...])
                      + (dis * dis) * h_ref[...] + b_ref[...])

    return pl.pallas_call(
        body,
        grid=(n // bn,),
        in_specs=[pl.BlockSpec((bn, d), lambda i: (i, 0)),
                  pl.BlockSpec((bn, d), lambda i: (i, 0)),
                  pl.BlockSpec((bn, d), lambda i: (i, 0)),
                  pl.BlockSpec((NW, bn), lambda i: (0, i)),
                  pl.BlockSpec((1, d), lambda i: (0, 0))],
        out_specs=pl.BlockSpec((bn, d), lambda i: (i, 0)),
        out_shape=jax.ShapeDtypeStruct((n, d), jnp.float32),
    )(acc0, acc1, h, degp, b.reshape(1, d))


def _sc_degree(dst_p, zeros_tab, n_nodes):
    ep = dst_p.shape[0]
    per_w = ep // NW
    per_sub = _ceil_to((n_nodes + 1 + NS - 1) // NS, CHUNK)
    n_pad = per_sub * NS
    mesh = plsc.VectorSubcoreMesh(core_axis_name="c", subcore_axis_name="s")

    cp = pltpu.CompilerParams()
    if "needs_layout_passes" in pltpu.CompilerParams.__dataclass_fields__:
        cp = dataclasses.replace(cp, needs_layout_passes=False)

    @functools.partial(
        pl.kernel, mesh=mesh,
        out_type=jax.ShapeDtypeStruct((NW, n_pad), jnp.float32),
        compiler_params=cp,
        scratch_types=[
            pltpu.VMEM((per_w,), jnp.int32),
            pltpu.VMEM((n_pad,), jnp.float32),
        ])
    def deg_kernel(dst_hbm, zeros_hbm, out_hbm, idx_v, tab_v):
        cid = lax.axis_index("c")
        sid = lax.axis_index("s")
        wid = cid * NS + sid
        pltpu.sync_copy(dst_hbm.at[pl.ds(wid * per_w, per_w)], idx_v)
        pltpu.sync_copy(zeros_hbm, tab_v)
        ones16 = jnp.ones((16,), jnp.float32)

        @pl.loop(0, per_w // 16)
        def _(k):
            idx = idx_v[pl.ds(k * 16, 16)]
            plsc.addupdate_scatter(tab_v, [idx], ones16)

        pltpu.sync_copy(tab_v, out_hbm.at[wid])

    return deg_kernel(dst_p, zeros_tab)


def _sc_aggregate(hs, src_p, dst_p, zeros_agg, n_nodes):
    ep = src_p.shape[0]
    d = hs.shape[1]
    cpw = ep // (NW * CHUNK)
    per_w = cpw * CHUNK
    per_sub = _ceil_to((n_nodes + 1 + NS - 1) // NS, CHUNK)
    acc_rows = per_sub * NS
    mesh = plsc.VectorSubcoreMesh(core_axis_name="c", subcore_axis_name="s")

    half = cpw // 2

    @functools.partial(
        pl.kernel, mesh=mesh,
        out_type=jax.ShapeDtypeStruct((NC, acc_rows, d), jnp.float32),
        scratch_types=[
            pltpu.VMEM((half, CHUNK), jnp.int32),
            pltpu.VMEM((half, CHUNK), jnp.int32),
            pltpu.VMEM((CHUNK, d), jnp.float32),
            pltpu.VMEM((CHUNK, d), jnp.float32),
            pltpu.VMEM_SHARED((acc_rows, d), jnp.float32),
            pltpu.SemaphoreType.DMA,
            pltpu.SemaphoreType.DMA,
        ])
    def agg_kernel(hs_hbm, src_hbm, dst_hbm, zeros_hbm, out_hbm,
                   src_v, dst_v, rows_a, rows_b, acc_sh, sem_a, sem_b):
        cid = lax.axis_index("c")
        sid = lax.axis_index("s")
        wid = cid * NS + sid

        pltpu.sync_copy(zeros_hbm, acc_sh.at[pl.ds(sid * per_sub, per_sub)])

        plsc.subcore_barrier()

        def scatter(c, rows):
            pltpu.sync_copy(rows, acc_sh.at[dst_v.at[c]], add=True)

        for hlf in range(2):
            pltpu.sync_copy(src_hbm.at[pl.ds(wid * cpw + hlf * half, half)],
                            src_v)
            pltpu.sync_copy(dst_hbm.at[pl.ds(wid * cpw + hlf * half, half)],
                            dst_v)

            @pl.loop(0, half // 2)
            def _(i):
                ga = pltpu.async_copy(hs_hbm.at[src_v.at[2 * i]],
                                      rows_a, sem_a)
                gb = pltpu.async_copy(hs_hbm.at[src_v.at[2 * i + 1]],
                                      rows_b, sem_b)
                ga.wait()
                scatter(2 * i, rows_a)
                gb.wait()
                scatter(2 * i + 1, rows_b)

        plsc.subcore_barrier()
        pltpu.sync_copy(acc_sh.at[pl.ds(sid * per_sub, per_sub)],
                        out_hbm.at[cid, pl.ds(sid * per_sub, per_sub)])

    return agg_kernel(hs, src_p.reshape(-1, CHUNK), dst_p.reshape(-1, CHUNK),
                      zeros_agg)


def kernel(x, edge_index, W, b):
    n, _ = x.shape
    e = edge_index.shape[1]
    src = edge_index[0].astype(jnp.int32)
    dst = edge_index[1].astype(jnp.int32)

    ep = _ceil_to(e, NW * CHUNK * 8)
    per_sub = _ceil_to((n + 1 + NS - 1) // NS, CHUNK)
    acc_rows = per_sub * NS
    pad_dst = n + jnp.arange(ep - e, dtype=jnp.int32) % (acc_rows - n)
    src_p = jnp.concatenate([src, jnp.zeros((ep - e,), jnp.int32)])
    dst_p = jnp.concatenate([dst, pad_dst])

    zeros_tab = jnp.zeros((acc_rows,), jnp.float32)
    zeros_agg = jnp.zeros((per_sub, W.shape[0]), jnp.float32)

    x_pad = jnp.concatenate(
        [x, jnp.zeros((acc_rows - n, x.shape[1]), x.dtype)])

    degp = _sc_degree(dst_p, zeros_tab, n)
    h = _tc_matmul(x_pad, W)
    hs = _tc_prescale(h, degp)
    acc_part = _sc_aggregate(hs, src_p, dst_p, zeros_agg, n)
    out = _tc_final(acc_part[0], acc_part[1], h, degp, b)
    return out[:n]
